# Initial kernel scaffold; baseline (speedup 1.0000x reference)
import jax
import jax.numpy as jnp
from jax.experimental import pallas as pl


def kernel(z, codebook):
    raise NotImplementedError("write your pallas kernel here")


# speedup vs baseline: 1.3606x; 1.3606x over previous
import functools

import jax
import jax.numpy as jnp
from jax import lax
from jax.experimental import pallas as pl
from jax.experimental.pallas import tpu as pltpu
from jax.experimental.pallas import tpu_sc as plsc

_BR = 256


def _dist_argmin_body(x_ref, cb_ref, cnorm_ref, idx_ref, mind_ref):
    x = x_ref[...]
    xsq = jnp.sum(x * x, axis=1, keepdims=True)
    scores = lax.dot_general(
        x, cb_ref[...], (((1,), (1,)), ((), ())),
        preferred_element_type=jnp.float32)
    dist = xsq - 2.0 * scores + cnorm_ref[...]
    idx_ref[0, 0, :] = jnp.argmin(dist, axis=1).astype(jnp.int32)
    mind_ref[0, 0, :] = jnp.min(dist, axis=1)


def _dist_argmin(flat, codebook, cnorm):
    n, d = flat.shape
    k = codebook.shape[0]
    nb = n // _BR
    return pl.pallas_call(
        _dist_argmin_body,
        grid=(nb,),
        in_specs=[
            pl.BlockSpec((_BR, d), lambda i: (i, 0)),
            pl.BlockSpec((k, d), lambda i: (0, 0)),
            pl.BlockSpec((1, k), lambda i: (0, 0)),
        ],
        out_specs=[
            pl.BlockSpec((1, 1, _BR), lambda i: (i, 0, 0)),
            pl.BlockSpec((1, 1, _BR), lambda i: (i, 0, 0)),
        ],
        out_shape=[
            jax.ShapeDtypeStruct((nb, 1, _BR), jnp.int32),
            jax.ShapeDtypeStruct((nb, 1, _BR), jnp.float32),
        ],
    )(flat, codebook, cnorm)


_CR = 128


def _make_sc_gather(n, d, k):
    info = plsc.get_sparse_core_info()
    nw = info.num_cores * info.num_subcores
    bpw = n // nw
    nchunks = bpw // _CR
    mesh = plsc.VectorSubcoreMesh(core_axis_name="c", subcore_axis_name="s")

    @functools.partial(
        pl.kernel,
        mesh=mesh,
        out_type=jax.ShapeDtypeStruct((n, d), jnp.float32),
        scratch_types=[
            pltpu.VMEM((bpw,), jnp.int32),
            pltpu.VMEM((_CR, d), jnp.float32),
            pltpu.SemaphoreType.DMA,
        ],
    )
    def gather_rows(table_hbm, idx_hbm, out_hbm, idx_v, rows_v, sem):
        wid = lax.axis_index("s") * info.num_cores + lax.axis_index("c")
        base = wid * bpw
        pltpu.sync_copy(idx_hbm.at[pl.ds(base, bpw)], idx_v)
        for c in range(nchunks):
            pltpu.async_copy(
                table_hbm.at[idx_v.at[pl.ds(c * _CR, _CR)]], rows_v, sem
            ).wait()
            pltpu.sync_copy(rows_v, out_hbm.at[pl.ds(base + c * _CR, _CR)])

    return gather_rows


def kernel(z, codebook):
    b, t, d = z.shape
    n = b * t
    k = codebook.shape[0]
    flat = z.reshape(-1, d)
    cnorm = jnp.sum(codebook ** 2, axis=1).reshape(1, k)
    idx3, mind3 = _dist_argmin(flat, codebook, cnorm)
    idx = idx3.reshape(-1)
    loss = 1.25 * (jnp.sum(mind3) / (n * d))
    q = _make_sc_gather(n, d, k)(codebook, idx)
    return q.reshape(b, t, d), idx.reshape(b, t), loss
